# baseline (speedup 1.0000x reference)
import jax
import jax.numpy as jnp
from jax import lax
from jax.experimental import pallas as pl
from jax.experimental.pallas import tpu as pltpu
from jax.experimental.pallas import tpu_sc as plsc

N = 100000
E = 3200000
K = 1000
LAYERS = 3

CH = 128
NCHUNK = E // CH
NTILES = 32
NSUPER = NCHUNK // 8
BASE_SUPER = NSUPER // NTILES
EXTRA = NSUPER - BASE_SUPER * NTILES
WINC = 16
NWIN = (BASE_SUPER * 8) // WINC
TAILC = BASE_SUPER * 8 - NWIN * WINC
SL = 6256
LAST_BASE = N - SL
SUBS = (1280, 1280, 1280, 1280, 1136)
SBUF = 1280
KPAD = 1024

MAGA = (0x1A2B3C01, 0x1A2B3C02, 0x1A2B3C03)
MAGC = (0x4D5E6F01, 0x4D5E6F02, 0x4D5E6F03)


def _tc_w_body(a_ref, o_ref):
    o_ref[...] = jnp.log1p(a_ref[...])


def _tc_h0_body(x_ref, r_ref, o_ref):
    o_ref[...] = x_ref[...] * r_ref[...]


def _sc_mega_body(h0_hbm, col_hbm, row_hbm, w_hbm, dmi_hbm, dmv_hbm, fcw_hbm,
                  out_hbm, p0_hbm, p1_hbm, hc_hbm, flag_hbm,
                  h_new_s, h_local, bufA, bufB,
                  col_b, row_b, w_b, val_b,
                  col_c, row_c, w_c, val_c,
                  dmi_b, g0, g1, dv, fw, ob, fwb, frb,
                  ssem, lsem0, lsem1, gsem):
    c = lax.axis_index("c")
    s = lax.axis_index("s")
    wid = c * 16 + s
    base = jnp.minimum(s * SL, LAST_BASE)

    def _publish(rowix, magic):
        @pl.when(s == 0)
        def _():
            fwb[...] = jnp.full((16,), magic, jnp.int32)
            pltpu.sync_copy(fwb, flag_hbm.at[rowix])

    def _poll(rowix, magic):
        @pl.when(s == 0)
        def _():
            def _cond(v):
                return v != magic

            def _body(v):
                pltpu.sync_copy(flag_hbm.at[rowix], frb)
                return frb[...][0]

            lax.while_loop(_cond, _body, jnp.int32(magic) ^ 1)

    cstart = wid * BASE_SUPER * 8
    bufs = ((col_b, row_b, w_b, val_b), (col_c, row_c, w_c, val_c))
    lsems = (lsem0, lsem1)

    def _start_loads(p, k):
        cb, rb, wb, _ = bufs[p]
        r0 = cstart + k * WINC
        ls = lsems[p]
        pltpu.async_copy(col_hbm.at[pl.ds(r0, WINC)], cb, ls)
        pltpu.async_copy(row_hbm.at[pl.ds(r0, WINC)], rb, ls)
        pltpu.async_copy(w_hbm.at[pl.ds(r0, WINC)], wb, ls)

    def _wait_loads(p):
        cb, rb, wb, _ = bufs[p]
        ls = lsems[p]
        pltpu.make_async_copy(col_hbm.at[pl.ds(0, WINC)], cb, ls).wait()
        pltpu.make_async_copy(row_hbm.at[pl.ds(0, WINC)], rb, ls).wait()
        pltpu.make_async_copy(w_hbm.at[pl.ds(0, WINC)], wb, ls).wait()

    def _work(p, nch):
        cb, rb, wb, vb = bufs[p]

        def _mf(i, cy):
            for j in range(8):
                sl = pl.ds(j * 16, 16)
                vals = plsc.load_gather(h_local, [cb[i, sl]])
                vb[i, sl] = vals * wb[i, sl]
            pltpu.async_copy(vb.at[i], h_new_s.at[rb.at[i]], ssem, add=True)
            return cy

        lax.fori_loop(0, nch, _mf, 0)
        pltpu.make_async_copy(w_hbm.at[pl.ds(0, nch)], vb.at[pl.ds(0, nch)],
                              ssem).wait()

    for lyr in range(LAYERS):
        if lyr > 0:
            _poll(2 + (1 - c), MAGC[lyr - 1])
            plsc.subcore_barrier()

        off = base
        for size in SUBS:
            sla = pl.ds(0, size)
            odst = pl.ds(off, size)
            if lyr == 0:
                pltpu.sync_copy(h0_hbm.at[odst], bufA.at[sla])
            else:
                pltpu.sync_copy(p0_hbm.at[odst], bufA.at[sla])
                pltpu.sync_copy(p1_hbm.at[odst], bufB.at[sla])

            def _add(j, carry):
                sl = pl.ds(j * 16, 16)
                if lyr > 0:
                    bufA[sl] = bufA[sl] + bufB[sl]
                bufB[sl] = jnp.zeros((16,), jnp.float32)
                return carry

            lax.fori_loop(0, size // 16, _add, 0)
            pltpu.sync_copy(bufA.at[sla], hc_hbm.at[odst])
            pltpu.sync_copy(bufB.at[sla], h_new_s.at[odst])
            off = off + size

        plsc.subcore_barrier()
        _publish(c, MAGA[lyr])
        pltpu.sync_copy(hc_hbm, h_local)

        _start_loads(0, 0)

        def _pair(t, carry):
            k0 = 2 * t
            _wait_loads(0)
            _start_loads(1, k0 + 1)
            _work(0, WINC)
            _wait_loads(1)

            @pl.when(k0 + 2 < NWIN)
            def _sl():
                _start_loads(0, k0 + 2)

            _work(1, WINC)
            return carry

        lax.fori_loop(0, NWIN // 2, _pair, 0)

        def _do_window(r0, nch):
            sl_w = pl.ds(0, nch)
            pltpu.sync_copy(col_hbm.at[pl.ds(r0, nch)], col_b.at[sl_w])
            pltpu.sync_copy(row_hbm.at[pl.ds(r0, nch)], row_b.at[sl_w])
            pltpu.sync_copy(w_hbm.at[pl.ds(r0, nch)], w_b.at[sl_w])
            _work(0, nch)

        _do_window(cstart + NWIN * WINC, TAILC)

        @pl.when(wid < EXTRA)
        def _extra():
            _do_window((NTILES * BASE_SUPER + wid) * 8, 8)

        plsc.subcore_barrier()
        _poll(1 - c, MAGA[lyr])
        plsc.subcore_barrier()

        off = base
        for size in SUBS:
            sla = pl.ds(0, size)
            odst = pl.ds(off, size)
            pltpu.sync_copy(h_new_s.at[odst], bufA.at[sla])

            @pl.when(c == 0)
            def _w0():
                pltpu.sync_copy(bufA.at[sla], p0_hbm.at[odst])

            @pl.when(c == 1)
            def _w1():
                pltpu.sync_copy(bufA.at[sla], p1_hbm.at[odst])

            off = off + size

        plsc.subcore_barrier()
        _publish(2 + c, MAGC[lyr])

    _poll(2 + (1 - c), MAGC[LAYERS - 1])

    @pl.when((c == 0) & (s == 0))
    def _final():
        pltpu.sync_copy(dmi_hbm, dmi_b)
        pltpu.sync_copy(dmv_hbm, dv)
        pltpu.sync_copy(fcw_hbm, fw)

        def _fire(i, cy):
            pltpu.async_copy(p0_hbm.at[dmi_b.at[i]], g0.at[i], gsem)
            pltpu.async_copy(p1_hbm.at[dmi_b.at[i]], g1.at[i], gsem)
            return cy

        lax.fori_loop(0, KPAD // CH, _fire, 0)
        pltpu.make_async_copy(dmv_hbm, g0, gsem).wait()
        pltpu.make_async_copy(dmv_hbm, g1, gsem).wait()

        def _red(f, acc):
            i = f // 8
            sl = pl.ds((f % 8) * 16, 16)
            return acc + (g0[i, sl] + g1[i, sl]) * dv[i, sl] * fw[i, sl]

        acc = lax.fori_loop(0, (KPAD // CH) * 8, _red,
                            jnp.zeros((16,), jnp.float32))
        dnums = lax.GatherDimensionNumbers(
            offset_dims=(), collapsed_slice_dims=(0,), start_index_map=(0,))
        for shift in (8, 4, 2, 1):
            perm = lax.iota(jnp.int32, 16) ^ shift
            acc = acc + lax.gather(
                acc, perm[:, None], dnums, (1,),
                mode=lax.GatherScatterMode.PROMISE_IN_BOUNDS)
        ob[...] = acc
        pltpu.sync_copy(ob, out_hbm)


_sc_mesh = plsc.VectorSubcoreMesh(core_axis_name="c", subcore_axis_name="s")

_sc_mega = pl.kernel(
    _sc_mega_body,
    out_type=(jax.ShapeDtypeStruct((16,), jnp.float32),
              jax.ShapeDtypeStruct((N,), jnp.float32),
              jax.ShapeDtypeStruct((N,), jnp.float32),
              jax.ShapeDtypeStruct((N,), jnp.float32),
              jax.ShapeDtypeStruct((4, 16), jnp.int32)),
    mesh=_sc_mesh,
    compiler_params=pltpu.CompilerParams(needs_layout_passes=False),
    scratch_types=[
        pltpu.VMEM_SHARED((N,), jnp.float32),
        pltpu.VMEM((N,), jnp.float32),
        pltpu.VMEM((SBUF,), jnp.float32),
        pltpu.VMEM((SBUF,), jnp.float32),
        pltpu.VMEM((WINC, CH), jnp.int32),
        pltpu.VMEM((WINC, CH), jnp.int32),
        pltpu.VMEM((WINC, CH), jnp.float32),
        pltpu.VMEM((WINC, CH), jnp.float32),
        pltpu.VMEM((WINC, CH), jnp.int32),
        pltpu.VMEM((WINC, CH), jnp.int32),
        pltpu.VMEM((WINC, CH), jnp.float32),
        pltpu.VMEM((WINC, CH), jnp.float32),
        pltpu.VMEM((KPAD // CH, CH), jnp.int32),
        pltpu.VMEM((KPAD // CH, CH), jnp.float32),
        pltpu.VMEM((KPAD // CH, CH), jnp.float32),
        pltpu.VMEM((KPAD // CH, CH), jnp.float32),
        pltpu.VMEM((KPAD // CH, CH), jnp.float32),
        pltpu.VMEM((16,), jnp.float32),
        pltpu.VMEM((16,), jnp.int32),
        pltpu.VMEM((16,), jnp.int32),
        pltpu.SemaphoreType.DMA,
        pltpu.SemaphoreType.DMA,
        pltpu.SemaphoreType.DMA,
        pltpu.SemaphoreType.DMA,
    ],
)


def kernel(x, edge_index, adj_data, retina_scale, dm_idx, dm_vals, fc_w, fc_b):
    col2d = edge_index[1].reshape(NCHUNK, CH)
    row2d = edge_index[0].reshape(NCHUNK, CH)

    w2d = pl.pallas_call(
        _tc_w_body,
        grid=(125,),
        in_specs=[pl.BlockSpec((NCHUNK // 125, CH), lambda i: (i, 0))],
        out_specs=pl.BlockSpec((NCHUNK // 125, CH), lambda i: (i, 0)),
        out_shape=jax.ShapeDtypeStruct((NCHUNK, CH), jnp.float32),
    )(adj_data.reshape(NCHUNK, CH))

    h0 = pl.pallas_call(
        _tc_h0_body,
        out_shape=jax.ShapeDtypeStruct((N,), jnp.float32),
    )(x.reshape(N), retina_scale)

    dmi = jnp.zeros((KPAD,), jnp.int32).at[:K].set(dm_idx).reshape(KPAD // CH, CH)
    dmv = jnp.zeros((KPAD,), jnp.float32).at[:K].set(dm_vals).reshape(KPAD // CH, CH)
    fcw = jnp.zeros((KPAD,), jnp.float32).at[:K].set(fc_w[0]).reshape(KPAD // CH, CH)

    out_vec, _p0, _p1, _hc, _fl = _sc_mega(h0, col2d, row2d, w2d, dmi, dmv, fcw)
    return out_vec[0:1] + fc_b


# speedup vs baseline: 1.0036x; 1.0036x over previous
import jax
import jax.numpy as jnp
from jax import lax
from jax.experimental import pallas as pl
from jax.experimental.pallas import tpu as pltpu
from jax.experimental.pallas import tpu_sc as plsc

N = 100000
E = 3200000
K = 1000
LAYERS = 3

CH = 128
NCHUNK = E // CH
NTILES = 32
NSUPER = NCHUNK // 8
BASE_SUPER = NSUPER // NTILES
EXTRA = NSUPER - BASE_SUPER * NTILES
WINC = 16
NWIN = (BASE_SUPER * 8) // WINC
TAILC = BASE_SUPER * 8 - NWIN * WINC
SL = 6256
LAST_BASE = N - SL
SUBS = (1280, 1280, 1280, 1280, 1136)
SBUF = 1280
KPAD = 1024

MAGA = (0x1A2B3C01, 0x1A2B3C02, 0x1A2B3C03)
MAGC = (0x4D5E6F01, 0x4D5E6F02, 0x4D5E6F03)


def _tc_w_body(a_ref, o_ref):
    o_ref[...] = jnp.log1p(a_ref[...])


def _tc_h0_body(x_ref, r_ref, o_ref):
    o_ref[...] = x_ref[...] * r_ref[...]


def _sc_mega_body(h0_hbm, col_hbm, row_hbm, w_hbm, dmi_hbm, dmv_hbm, fcw_hbm,
                  out_hbm, p0_hbm, p1_hbm, hc_hbm, flag_hbm,
                  h_new_s, h_local, bufA, bufB,
                  col_b, row_b, w_b,
                  col_c, row_c, w_c,
                  dmi_b, g0, g1, dv, fw, ob, fwb, frb,
                  ssem, lsem0, lsem1, gsem):
    c = lax.axis_index("c")
    s = lax.axis_index("s")
    wid = c * 16 + s
    base = jnp.minimum(s * SL, LAST_BASE)

    def _publish(rowix, magic):
        @pl.when(s == 0)
        def _():
            fwb[...] = jnp.full((16,), magic, jnp.int32)
            pltpu.sync_copy(fwb, flag_hbm.at[rowix])

    def _poll(rowix, magic):
        @pl.when(s == 0)
        def _():
            def _cond(v):
                return v != magic

            def _body(v):
                pltpu.sync_copy(flag_hbm.at[rowix], frb)
                return frb[...][0]

            lax.while_loop(_cond, _body, jnp.int32(magic) ^ 1)

    cstart = wid * BASE_SUPER * 8
    bufs = ((col_b, row_b, w_b), (col_c, row_c, w_c))
    lsems = (lsem0, lsem1)
    ssems = (ssem, gsem)

    def _start_loads(p, k):
        cb, rb, wb = bufs[p]
        r0 = cstart + k * WINC
        ls = lsems[p]
        pltpu.async_copy(col_hbm.at[pl.ds(r0, WINC)], cb, ls)
        pltpu.async_copy(row_hbm.at[pl.ds(r0, WINC)], rb, ls)
        pltpu.async_copy(w_hbm.at[pl.ds(r0, WINC)], wb, ls)

    def _wait_loads(p):
        cb, rb, wb = bufs[p]
        ls = lsems[p]
        pltpu.make_async_copy(col_hbm.at[pl.ds(0, WINC)], cb, ls).wait()
        pltpu.make_async_copy(row_hbm.at[pl.ds(0, WINC)], rb, ls).wait()
        pltpu.make_async_copy(w_hbm.at[pl.ds(0, WINC)], wb, ls).wait()

    def _work(p, nch):
        cb, rb, wb = bufs[p]

        def _mf(i, cy):
            for j in range(8):
                sl = pl.ds(j * 16, 16)
                vals = plsc.load_gather(h_local, [cb[i, sl]])
                wb[i, sl] = vals * wb[i, sl]
            pltpu.async_copy(wb.at[i], h_new_s.at[rb.at[i]], ssems[p],
                             add=True)
            return cy

        lax.fori_loop(0, nch, _mf, 0)

    def _drain_scatters(p, nch):
        _, _, wb = bufs[p]
        pltpu.make_async_copy(w_hbm.at[pl.ds(0, nch)], wb.at[pl.ds(0, nch)],
                              ssems[p]).wait()

    for lyr in range(LAYERS):
        if lyr > 0:
            _poll(2 + (1 - c), MAGC[lyr - 1])
            plsc.subcore_barrier()

        off = base
        for size in SUBS:
            sla = pl.ds(0, size)
            odst = pl.ds(off, size)
            if lyr == 0:
                pltpu.sync_copy(h0_hbm.at[odst], bufA.at[sla])
            else:
                pltpu.sync_copy(p0_hbm.at[odst], bufA.at[sla])
                pltpu.sync_copy(p1_hbm.at[odst], bufB.at[sla])

            def _add(j, carry):
                sl = pl.ds(j * 16, 16)
                if lyr > 0:
                    bufA[sl] = bufA[sl] + bufB[sl]
                bufB[sl] = jnp.zeros((16,), jnp.float32)
                return carry

            lax.fori_loop(0, size // 16, _add, 0)
            pltpu.sync_copy(bufA.at[sla], hc_hbm.at[odst])
            pltpu.sync_copy(bufB.at[sla], h_new_s.at[odst])
            off = off + size

        plsc.subcore_barrier()
        _publish(c, MAGA[lyr])
        pltpu.sync_copy(hc_hbm, h_local)

        _start_loads(0, 0)

        def _pair(t, carry):
            k0 = 2 * t
            _wait_loads(0)

            @pl.when(t > 0)
            def _dp1():
                _drain_scatters(1, WINC)

            _start_loads(1, k0 + 1)
            _work(0, WINC)
            _wait_loads(1)
            _drain_scatters(0, WINC)

            @pl.when(k0 + 2 < NWIN)
            def _sl():
                _start_loads(0, k0 + 2)

            _work(1, WINC)
            return carry

        lax.fori_loop(0, NWIN // 2, _pair, 0)
        _drain_scatters(1, WINC)

        def _do_window(r0, nch):
            sl_w = pl.ds(0, nch)
            pltpu.sync_copy(col_hbm.at[pl.ds(r0, nch)], col_b.at[sl_w])
            pltpu.sync_copy(row_hbm.at[pl.ds(r0, nch)], row_b.at[sl_w])
            pltpu.sync_copy(w_hbm.at[pl.ds(r0, nch)], w_b.at[sl_w])
            _work(0, nch)
            _drain_scatters(0, nch)

        _do_window(cstart + NWIN * WINC, TAILC)

        @pl.when(wid < EXTRA)
        def _extra():
            _do_window((NTILES * BASE_SUPER + wid) * 8, 8)

        plsc.subcore_barrier()
        _poll(1 - c, MAGA[lyr])
        plsc.subcore_barrier()

        off = base
        for size in SUBS:
            sla = pl.ds(0, size)
            odst = pl.ds(off, size)
            pltpu.sync_copy(h_new_s.at[odst], bufA.at[sla])

            @pl.when(c == 0)
            def _w0():
                pltpu.sync_copy(bufA.at[sla], p0_hbm.at[odst])

            @pl.when(c == 1)
            def _w1():
                pltpu.sync_copy(bufA.at[sla], p1_hbm.at[odst])

            off = off + size

        plsc.subcore_barrier()
        _publish(2 + c, MAGC[lyr])

    _poll(2 + (1 - c), MAGC[LAYERS - 1])

    @pl.when((c == 0) & (s == 0))
    def _final():
        pltpu.sync_copy(dmi_hbm, dmi_b)
        pltpu.sync_copy(dmv_hbm, dv)
        pltpu.sync_copy(fcw_hbm, fw)

        def _fire(i, cy):
            pltpu.async_copy(p0_hbm.at[dmi_b.at[i]], g0.at[i], gsem)
            pltpu.async_copy(p1_hbm.at[dmi_b.at[i]], g1.at[i], gsem)
            return cy

        lax.fori_loop(0, KPAD // CH, _fire, 0)
        pltpu.make_async_copy(dmv_hbm, g0, gsem).wait()
        pltpu.make_async_copy(dmv_hbm, g1, gsem).wait()

        def _red(f, acc):
            i = f // 8
            sl = pl.ds((f % 8) * 16, 16)
            return acc + (g0[i, sl] + g1[i, sl]) * dv[i, sl] * fw[i, sl]

        acc = lax.fori_loop(0, (KPAD // CH) * 8, _red,
                            jnp.zeros((16,), jnp.float32))
        dnums = lax.GatherDimensionNumbers(
            offset_dims=(), collapsed_slice_dims=(0,), start_index_map=(0,))
        for shift in (8, 4, 2, 1):
            perm = lax.iota(jnp.int32, 16) ^ shift
            acc = acc + lax.gather(
                acc, perm[:, None], dnums, (1,),
                mode=lax.GatherScatterMode.PROMISE_IN_BOUNDS)
        ob[...] = acc
        pltpu.sync_copy(ob, out_hbm)


_sc_mesh = plsc.VectorSubcoreMesh(core_axis_name="c", subcore_axis_name="s")

_sc_mega = pl.kernel(
    _sc_mega_body,
    out_type=(jax.ShapeDtypeStruct((16,), jnp.float32),
              jax.ShapeDtypeStruct((N,), jnp.float32),
              jax.ShapeDtypeStruct((N,), jnp.float32),
              jax.ShapeDtypeStruct((N,), jnp.float32),
              jax.ShapeDtypeStruct((4, 16), jnp.int32)),
    mesh=_sc_mesh,
    compiler_params=pltpu.CompilerParams(needs_layout_passes=False),
    scratch_types=[
        pltpu.VMEM_SHARED((N,), jnp.float32),
        pltpu.VMEM((N,), jnp.float32),
        pltpu.VMEM((SBUF,), jnp.float32),
        pltpu.VMEM((SBUF,), jnp.float32),
        pltpu.VMEM((WINC, CH), jnp.int32),
        pltpu.VMEM((WINC, CH), jnp.int32),
        pltpu.VMEM((WINC, CH), jnp.float32),
        pltpu.VMEM((WINC, CH), jnp.int32),
        pltpu.VMEM((WINC, CH), jnp.int32),
        pltpu.VMEM((WINC, CH), jnp.float32),
        pltpu.VMEM((KPAD // CH, CH), jnp.int32),
        pltpu.VMEM((KPAD // CH, CH), jnp.float32),
        pltpu.VMEM((KPAD // CH, CH), jnp.float32),
        pltpu.VMEM((KPAD // CH, CH), jnp.float32),
        pltpu.VMEM((KPAD // CH, CH), jnp.float32),
        pltpu.VMEM((16,), jnp.float32),
        pltpu.VMEM((16,), jnp.int32),
        pltpu.VMEM((16,), jnp.int32),
        pltpu.SemaphoreType.DMA,
        pltpu.SemaphoreType.DMA,
        pltpu.SemaphoreType.DMA,
        pltpu.SemaphoreType.DMA,
    ],
)


def kernel(x, edge_index, adj_data, retina_scale, dm_idx, dm_vals, fc_w, fc_b):
    col2d = edge_index[1].reshape(NCHUNK, CH)
    row2d = edge_index[0].reshape(NCHUNK, CH)

    w2d = pl.pallas_call(
        _tc_w_body,
        grid=(125,),
        in_specs=[pl.BlockSpec((NCHUNK // 125, CH), lambda i: (i, 0))],
        out_specs=pl.BlockSpec((NCHUNK // 125, CH), lambda i: (i, 0)),
        out_shape=jax.ShapeDtypeStruct((NCHUNK, CH), jnp.float32),
    )(adj_data.reshape(NCHUNK, CH))

    h0 = pl.pallas_call(
        _tc_h0_body,
        out_shape=jax.ShapeDtypeStruct((N,), jnp.float32),
    )(x.reshape(N), retina_scale)

    dmi = jnp.zeros((KPAD,), jnp.int32).at[:K].set(dm_idx).reshape(KPAD // CH, CH)
    dmv = jnp.zeros((KPAD,), jnp.float32).at[:K].set(dm_vals).reshape(KPAD // CH, CH)
    fcw = jnp.zeros((KPAD,), jnp.float32).at[:K].set(fc_w[0]).reshape(KPAD // CH, CH)

    out_vec, _p0, _p1, _hc, _fl = _sc_mega(h0, col2d, row2d, w2d, dmi, dmv, fcw)
    return out_vec[0:1] + fc_b


# speedup vs baseline: 1.0040x; 1.0004x over previous
import jax
import jax.numpy as jnp
from jax import lax
from jax.experimental import pallas as pl
from jax.experimental.pallas import tpu as pltpu
from jax.experimental.pallas import tpu_sc as plsc

N = 100000
E = 3200000
K = 1000
LAYERS = 3

CH = 128
NCHUNK = E // CH
NTILES = 32
NSUPER = NCHUNK // 8
BASE_SUPER = NSUPER // NTILES
EXTRA = NSUPER - BASE_SUPER * NTILES
WINC = 16
NWIN = (BASE_SUPER * 8) // WINC
TAILC = BASE_SUPER * 8 - NWIN * WINC
SL = 6256
LAST_BASE = N - SL
SUBS = (1280, 1280, 1280, 1280, 1136)
SBUF = 1280
KPAD = 1024

MAGA = (0x1A2B3C01, 0x1A2B3C02, 0x1A2B3C03)
MAGC = (0x4D5E6F01, 0x4D5E6F02, 0x4D5E6F03)


def _tc_fmt_body(e_ref, a_ref, w_ref, col_ref, row_ref):
    blk = a_ref.shape[0] // CH
    w_ref[...] = jnp.log1p(a_ref[...]).reshape(blk, CH)
    row_ref[...] = e_ref[0, :].reshape(blk, CH)
    col_ref[...] = e_ref[1, :].reshape(blk, CH)


def _tc_h0_body(x_ref, r_ref, o_ref):
    o_ref[...] = x_ref[...] * r_ref[...]


def _sc_mega_body(h0_hbm, col_hbm, row_hbm, w_hbm, dmi_hbm, dmv_hbm, fcw_hbm,
                  out_hbm, p0_hbm, p1_hbm, hc_hbm, flag_hbm,
                  h_new_s, h_local, bufA, bufB,
                  col_b, row_b, w_b,
                  col_c, row_c, w_c,
                  dmi_b, g0, g1, dv, fw, ob, fwb, frb,
                  ssem, lsem0, lsem1, gsem):
    c = lax.axis_index("c")
    s = lax.axis_index("s")
    wid = c * 16 + s
    base = jnp.minimum(s * SL, LAST_BASE)

    def _publish(rowix, magic):
        @pl.when(s == 0)
        def _():
            fwb[...] = jnp.full((16,), magic, jnp.int32)
            pltpu.sync_copy(fwb, flag_hbm.at[rowix])

    def _poll(rowix, magic):
        @pl.when(s == 0)
        def _():
            def _cond(v):
                return v != magic

            def _body(v):
                pltpu.sync_copy(flag_hbm.at[rowix], frb)
                return frb[...][0]

            lax.while_loop(_cond, _body, jnp.int32(magic) ^ 1)

    cstart = wid * BASE_SUPER * 8
    bufs = ((col_b, row_b, w_b), (col_c, row_c, w_c))
    lsems = (lsem0, lsem1)
    ssems = (ssem, gsem)

    def _start_loads(p, k):
        cb, rb, wb = bufs[p]
        r0 = cstart + k * WINC
        ls = lsems[p]
        pltpu.async_copy(col_hbm.at[pl.ds(r0, WINC)], cb, ls)
        pltpu.async_copy(row_hbm.at[pl.ds(r0, WINC)], rb, ls)
        pltpu.async_copy(w_hbm.at[pl.ds(r0, WINC)], wb, ls)

    def _wait_loads(p):
        cb, rb, wb = bufs[p]
        ls = lsems[p]
        pltpu.make_async_copy(col_hbm.at[pl.ds(0, WINC)], cb, ls).wait()
        pltpu.make_async_copy(row_hbm.at[pl.ds(0, WINC)], rb, ls).wait()
        pltpu.make_async_copy(w_hbm.at[pl.ds(0, WINC)], wb, ls).wait()

    def _work(p, nch):
        cb, rb, wb = bufs[p]

        def _mf(i, cy):
            for j in range(8):
                sl = pl.ds(j * 16, 16)
                vals = plsc.load_gather(h_local, [cb[i, sl]])
                wb[i, sl] = vals * wb[i, sl]
            pltpu.async_copy(wb.at[i], h_new_s.at[rb.at[i]], ssems[p],
                             add=True)
            return cy

        lax.fori_loop(0, nch, _mf, 0)

    def _drain_scatters(p, nch):
        _, _, wb = bufs[p]
        pltpu.make_async_copy(w_hbm.at[pl.ds(0, nch)], wb.at[pl.ds(0, nch)],
                              ssems[p]).wait()

    for lyr in range(LAYERS):
        if lyr > 0:
            _poll(2 + (1 - c), MAGC[lyr - 1])
            plsc.subcore_barrier()

        off = base
        for size in SUBS:
            sla = pl.ds(0, size)
            odst = pl.ds(off, size)
            if lyr == 0:
                pltpu.sync_copy(h0_hbm.at[odst], bufA.at[sla])
            else:
                pltpu.sync_copy(p0_hbm.at[odst], bufA.at[sla])
                pltpu.sync_copy(p1_hbm.at[odst], bufB.at[sla])

            def _add(j, carry):
                sl = pl.ds(j * 16, 16)
                if lyr > 0:
                    bufA[sl] = bufA[sl] + bufB[sl]
                bufB[sl] = jnp.zeros((16,), jnp.float32)
                return carry

            lax.fori_loop(0, size // 16, _add, 0)
            pltpu.sync_copy(bufA.at[sla], hc_hbm.at[odst])
            pltpu.sync_copy(bufB.at[sla], h_new_s.at[odst])
            off = off + size

        plsc.subcore_barrier()
        _publish(c, MAGA[lyr])
        pltpu.sync_copy(hc_hbm, h_local)

        _start_loads(0, 0)

        def _pair(t, carry):
            k0 = 2 * t
            _wait_loads(0)

            @pl.when(t > 0)
            def _dp1():
                _drain_scatters(1, WINC)

            _start_loads(1, k0 + 1)
            _work(0, WINC)
            _wait_loads(1)
            _drain_scatters(0, WINC)

            @pl.when(k0 + 2 < NWIN)
            def _sl():
                _start_loads(0, k0 + 2)

            _work(1, WINC)
            return carry

        lax.fori_loop(0, NWIN // 2, _pair, 0)
        _drain_scatters(1, WINC)

        def _do_window(r0, nch):
            sl_w = pl.ds(0, nch)
            pltpu.sync_copy(col_hbm.at[pl.ds(r0, nch)], col_b.at[sl_w])
            pltpu.sync_copy(row_hbm.at[pl.ds(r0, nch)], row_b.at[sl_w])
            pltpu.sync_copy(w_hbm.at[pl.ds(r0, nch)], w_b.at[sl_w])
            _work(0, nch)
            _drain_scatters(0, nch)

        _do_window(cstart + NWIN * WINC, TAILC)

        @pl.when(wid < EXTRA)
        def _extra():
            _do_window((NTILES * BASE_SUPER + wid) * 8, 8)

        plsc.subcore_barrier()
        _poll(1 - c, MAGA[lyr])
        plsc.subcore_barrier()

        off = base
        for size in SUBS:
            sla = pl.ds(0, size)
            odst = pl.ds(off, size)
            pltpu.sync_copy(h_new_s.at[odst], bufA.at[sla])

            @pl.when(c == 0)
            def _w0():
                pltpu.sync_copy(bufA.at[sla], p0_hbm.at[odst])

            @pl.when(c == 1)
            def _w1():
                pltpu.sync_copy(bufA.at[sla], p1_hbm.at[odst])

            off = off + size

        plsc.subcore_barrier()
        _publish(2 + c, MAGC[lyr])

    _poll(2 + (1 - c), MAGC[LAYERS - 1])

    @pl.when((c == 0) & (s == 0))
    def _final():
        pltpu.sync_copy(dmi_hbm, dmi_b)
        pltpu.sync_copy(dmv_hbm, dv)
        pltpu.sync_copy(fcw_hbm, fw)

        def _fire(i, cy):
            pltpu.async_copy(p0_hbm.at[dmi_b.at[i]], g0.at[i], gsem)
            pltpu.async_copy(p1_hbm.at[dmi_b.at[i]], g1.at[i], gsem)
            return cy

        lax.fori_loop(0, KPAD // CH, _fire, 0)
        pltpu.make_async_copy(dmv_hbm, g0, gsem).wait()
        pltpu.make_async_copy(dmv_hbm, g1, gsem).wait()

        def _red(f, acc):
            i = f // 8
            sl = pl.ds((f % 8) * 16, 16)
            return acc + (g0[i, sl] + g1[i, sl]) * dv[i, sl] * fw[i, sl]

        acc = lax.fori_loop(0, (KPAD // CH) * 8, _red,
                            jnp.zeros((16,), jnp.float32))
        dnums = lax.GatherDimensionNumbers(
            offset_dims=(), collapsed_slice_dims=(0,), start_index_map=(0,))
        for shift in (8, 4, 2, 1):
            perm = lax.iota(jnp.int32, 16) ^ shift
            acc = acc + lax.gather(
                acc, perm[:, None], dnums, (1,),
                mode=lax.GatherScatterMode.PROMISE_IN_BOUNDS)
        ob[...] = acc
        pltpu.sync_copy(ob, out_hbm)


_sc_mesh = plsc.VectorSubcoreMesh(core_axis_name="c", subcore_axis_name="s")

_sc_mega = pl.kernel(
    _sc_mega_body,
    out_type=(jax.ShapeDtypeStruct((16,), jnp.float32),
              jax.ShapeDtypeStruct((N,), jnp.float32),
              jax.ShapeDtypeStruct((N,), jnp.float32),
              jax.ShapeDtypeStruct((N,), jnp.float32),
              jax.ShapeDtypeStruct((4, 16), jnp.int32)),
    mesh=_sc_mesh,
    compiler_params=pltpu.CompilerParams(needs_layout_passes=False),
    scratch_types=[
        pltpu.VMEM_SHARED((N,), jnp.float32),
        pltpu.VMEM((N,), jnp.float32),
        pltpu.VMEM((SBUF,), jnp.float32),
        pltpu.VMEM((SBUF,), jnp.float32),
        pltpu.VMEM((WINC, CH), jnp.int32),
        pltpu.VMEM((WINC, CH), jnp.int32),
        pltpu.VMEM((WINC, CH), jnp.float32),
        pltpu.VMEM((WINC, CH), jnp.int32),
        pltpu.VMEM((WINC, CH), jnp.int32),
        pltpu.VMEM((WINC, CH), jnp.float32),
        pltpu.VMEM((KPAD // CH, CH), jnp.int32),
        pltpu.VMEM((KPAD // CH, CH), jnp.float32),
        pltpu.VMEM((KPAD // CH, CH), jnp.float32),
        pltpu.VMEM((KPAD // CH, CH), jnp.float32),
        pltpu.VMEM((KPAD // CH, CH), jnp.float32),
        pltpu.VMEM((16,), jnp.float32),
        pltpu.VMEM((16,), jnp.int32),
        pltpu.VMEM((16,), jnp.int32),
        pltpu.SemaphoreType.DMA,
        pltpu.SemaphoreType.DMA,
        pltpu.SemaphoreType.DMA,
        pltpu.SemaphoreType.DMA,
    ],
)


def kernel(x, edge_index, adj_data, retina_scale, dm_idx, dm_vals, fc_w, fc_b):
    grid = 125
    eblk = E // grid
    rblk = NCHUNK // grid
    w2d, col2d, row2d = pl.pallas_call(
        _tc_fmt_body,
        grid=(grid,),
        in_specs=[pl.BlockSpec((2, eblk), lambda i: (0, i)),
                  pl.BlockSpec((eblk,), lambda i: (i,))],
        out_specs=[pl.BlockSpec((rblk, CH), lambda i: (i, 0)),
                   pl.BlockSpec((rblk, CH), lambda i: (i, 0)),
                   pl.BlockSpec((rblk, CH), lambda i: (i, 0))],
        out_shape=[jax.ShapeDtypeStruct((NCHUNK, CH), jnp.float32),
                   jax.ShapeDtypeStruct((NCHUNK, CH), jnp.int32),
                   jax.ShapeDtypeStruct((NCHUNK, CH), jnp.int32)],
    )(edge_index, adj_data)

    h0 = pl.pallas_call(
        _tc_h0_body,
        out_shape=jax.ShapeDtypeStruct((N,), jnp.float32),
    )(x.reshape(N), retina_scale)

    dmi = jnp.zeros((KPAD,), jnp.int32).at[:K].set(dm_idx).reshape(KPAD // CH, CH)
    dmv = jnp.zeros((KPAD,), jnp.float32).at[:K].set(dm_vals).reshape(KPAD // CH, CH)
    fcw = jnp.zeros((KPAD,), jnp.float32).at[:K].set(fc_w[0]).reshape(KPAD // CH, CH)

    out_vec, _p0, _p1, _hc, _fl = _sc_mega(h0, col2d, row2d, w2d, dmi, dmv, fcw)
    return out_vec[0:1] + fc_b


# speedup vs baseline: 1.0251x; 1.0210x over previous
import jax
import jax.numpy as jnp
from jax import lax
from jax.experimental import pallas as pl
from jax.experimental.pallas import tpu as pltpu
from jax.experimental.pallas import tpu_sc as plsc

N = 100000
E = 3200000
K = 1000
LAYERS = 3

CH = 128
NCHUNK = E // CH
NTILES = 32
NSUPER = NCHUNK // 8
BASE_SUPER = NSUPER // NTILES
EXTRA = NSUPER - BASE_SUPER * NTILES
WINC = 16
NWIN = (BASE_SUPER * 8) // WINC
TAILC = BASE_SUPER * 8 - NWIN * WINC
SL = 6256
LAST_BASE = N - SL
SLA = 3200
SLB = SL - SLA
KPAD = 1024


def _tc_fmt_body(e_ref, a_ref, w_ref, col_ref, row_ref):
    blk = a_ref.shape[0] // CH
    w_ref[...] = jnp.log1p(a_ref[...]).reshape(blk, CH)
    row_ref[...] = e_ref[0, :].reshape(blk, CH)
    col_ref[...] = e_ref[1, :].reshape(blk, CH)


def _tc_h0_body(x_ref, r_ref, o_ref):
    o_ref[...] = x_ref[...] * r_ref[...]


def _sc_layer_body(h0_hbm, h1_hbm, col_hbm, row_hbm, w_hbm,
                   p0_hbm, p1_hbm, hc_hbm,
                   h_new_s, h_local, bufA, bufB,
                   col_b, row_b, w_b,
                   col_c, row_c, w_c,
                   ssem0, ssem1, lsem0, lsem1):
    c = lax.axis_index("c")
    s = lax.axis_index("s")
    wid = c * 16 + s

    base = jnp.minimum(s * SL, LAST_BASE)

    def _combine(off, size, n16):
        sla = pl.ds(0, size)
        pltpu.sync_copy(h0_hbm.at[pl.ds(off, size)], bufA.at[sla])
        pltpu.sync_copy(h1_hbm.at[pl.ds(off, size)], bufB.at[sla])

        def _add(j, carry):
            sl = pl.ds(j * 16, 16)
            bufA[sl] = bufA[sl] + bufB[sl]
            bufB[sl] = jnp.zeros((16,), jnp.float32)
            return carry

        lax.fori_loop(0, n16, _add, 0)
        pltpu.sync_copy(bufA.at[sla], hc_hbm.at[pl.ds(off, size)])
        pltpu.sync_copy(bufB.at[sla], h_new_s.at[pl.ds(off, size)])

    _combine(base, SLA, SLA // 16)
    _combine(base + SLA, SLB, SLB // 16)
    plsc.subcore_barrier()
    pltpu.sync_copy(hc_hbm, h_local)

    cstart = wid * BASE_SUPER * 8
    bufs = ((col_b, row_b, w_b), (col_c, row_c, w_c))
    lsems = (lsem0, lsem1)
    ssems = (ssem0, ssem1)

    def _start_loads(p, k):
        cb, rb, wb = bufs[p]
        r0 = cstart + k * WINC
        ls = lsems[p]
        pltpu.async_copy(col_hbm.at[pl.ds(r0, WINC)], cb, ls)
        pltpu.async_copy(row_hbm.at[pl.ds(r0, WINC)], rb, ls)
        pltpu.async_copy(w_hbm.at[pl.ds(r0, WINC)], wb, ls)

    def _wait_loads(p):
        cb, rb, wb = bufs[p]
        ls = lsems[p]
        pltpu.make_async_copy(col_hbm.at[pl.ds(0, WINC)], cb, ls).wait()
        pltpu.make_async_copy(row_hbm.at[pl.ds(0, WINC)], rb, ls).wait()
        pltpu.make_async_copy(w_hbm.at[pl.ds(0, WINC)], wb, ls).wait()

    def _work(p, nch):
        cb, rb, wb = bufs[p]

        def _mf(i, cy):
            for j in range(8):
                sl = pl.ds(j * 16, 16)
                vals = plsc.load_gather(h_local, [cb[i, sl]])
                wb[i, sl] = vals * wb[i, sl]
            pltpu.async_copy(wb.at[i], h_new_s.at[rb.at[i]], ssems[p],
                             add=True)
            return cy

        lax.fori_loop(0, nch, _mf, 0)

    def _drain_scatters(p, nch):
        _, _, wb = bufs[p]
        pltpu.make_async_copy(w_hbm.at[pl.ds(0, nch)], wb.at[pl.ds(0, nch)],
                              ssems[p]).wait()

    _start_loads(0, 0)

    def _pair(t, carry):
        k0 = 2 * t
        _wait_loads(0)

        @pl.when(t > 0)
        def _dp1():
            _drain_scatters(1, WINC)

        _start_loads(1, k0 + 1)
        _work(0, WINC)
        _wait_loads(1)
        _drain_scatters(0, WINC)

        @pl.when(k0 + 2 < NWIN)
        def _sl():
            _start_loads(0, k0 + 2)

        _work(1, WINC)
        return carry

    lax.fori_loop(0, NWIN // 2, _pair, 0)
    _drain_scatters(1, WINC)

    def _do_window(r0, nch):
        sl_w = pl.ds(0, nch)
        pltpu.sync_copy(col_hbm.at[pl.ds(r0, nch)], col_b.at[sl_w])
        pltpu.sync_copy(row_hbm.at[pl.ds(r0, nch)], row_b.at[sl_w])
        pltpu.sync_copy(w_hbm.at[pl.ds(r0, nch)], w_b.at[sl_w])
        _work(0, nch)
        _drain_scatters(0, nch)

    _do_window(cstart + NWIN * WINC, TAILC)

    @pl.when(wid < EXTRA)
    def _extra():
        _do_window((NTILES * BASE_SUPER + wid) * 8, 8)

    plsc.subcore_barrier()

    def _emit(off, size):
        sla = pl.ds(0, size)
        pltpu.sync_copy(h_new_s.at[pl.ds(off, size)], bufA.at[sla])

        @pl.when(c == 0)
        def _w0():
            pltpu.sync_copy(bufA.at[sla], p0_hbm.at[pl.ds(off, size)])

        @pl.when(c == 1)
        def _w1():
            pltpu.sync_copy(bufA.at[sla], p1_hbm.at[pl.ds(off, size)])

    _emit(base, SLA)
    _emit(base + SLA, SLB)


def _sc_final_body(p0_hbm, p1_hbm, dmi_hbm, dmv_hbm, fcw_hbm, out_hbm,
                   dmi_b, g0, g1, dv, fw, ob, gsem):
    c = lax.axis_index("c")
    s = lax.axis_index("s")

    @pl.when((c == 0) & (s == 0))
    def _work():
        pltpu.sync_copy(dmi_hbm, dmi_b)
        pltpu.sync_copy(dmv_hbm, dv)
        pltpu.sync_copy(fcw_hbm, fw)

        def _fire(i, cy):
            pltpu.async_copy(p0_hbm.at[dmi_b.at[i]], g0.at[i], gsem)
            pltpu.async_copy(p1_hbm.at[dmi_b.at[i]], g1.at[i], gsem)
            return cy

        lax.fori_loop(0, KPAD // CH, _fire, 0)
        pltpu.make_async_copy(dmv_hbm, g0, gsem).wait()
        pltpu.make_async_copy(dmv_hbm, g1, gsem).wait()

        def _red(f, acc):
            i = f // 8
            sl = pl.ds((f % 8) * 16, 16)
            return acc + (g0[i, sl] + g1[i, sl]) * dv[i, sl] * fw[i, sl]

        acc = lax.fori_loop(0, (KPAD // CH) * 8, _red,
                            jnp.zeros((16,), jnp.float32))
        dnums = lax.GatherDimensionNumbers(
            offset_dims=(), collapsed_slice_dims=(0,), start_index_map=(0,))
        for shift in (8, 4, 2, 1):
            perm = lax.iota(jnp.int32, 16) ^ shift
            acc = acc + lax.gather(
                acc, perm[:, None], dnums, (1,),
                mode=lax.GatherScatterMode.PROMISE_IN_BOUNDS)
        ob[...] = acc
        pltpu.sync_copy(ob, out_hbm)


_sc_mesh = plsc.VectorSubcoreMesh(core_axis_name="c", subcore_axis_name="s")

_sc_layer = pl.kernel(
    _sc_layer_body,
    out_type=(jax.ShapeDtypeStruct((N,), jnp.float32),
              jax.ShapeDtypeStruct((N,), jnp.float32),
              jax.ShapeDtypeStruct((N,), jnp.float32)),
    mesh=_sc_mesh,
    compiler_params=pltpu.CompilerParams(needs_layout_passes=False),
    scratch_types=[
        pltpu.VMEM_SHARED((N,), jnp.float32),
        pltpu.VMEM((N,), jnp.float32),
        pltpu.VMEM((SLA,), jnp.float32),
        pltpu.VMEM((SLA,), jnp.float32),
        pltpu.VMEM((WINC, CH), jnp.int32),
        pltpu.VMEM((WINC, CH), jnp.int32),
        pltpu.VMEM((WINC, CH), jnp.float32),
        pltpu.VMEM((WINC, CH), jnp.int32),
        pltpu.VMEM((WINC, CH), jnp.int32),
        pltpu.VMEM((WINC, CH), jnp.float32),
        pltpu.SemaphoreType.DMA,
        pltpu.SemaphoreType.DMA,
        pltpu.SemaphoreType.DMA,
        pltpu.SemaphoreType.DMA,
    ],
)

_sc_final = pl.kernel(
    _sc_final_body,
    out_type=jax.ShapeDtypeStruct((16,), jnp.float32),
    mesh=_sc_mesh,
    scratch_types=[
        pltpu.VMEM((KPAD // CH, CH), jnp.int32),
        pltpu.VMEM((KPAD // CH, CH), jnp.float32),
        pltpu.VMEM((KPAD // CH, CH), jnp.float32),
        pltpu.VMEM((KPAD // CH, CH), jnp.float32),
        pltpu.VMEM((KPAD // CH, CH), jnp.float32),
        pltpu.VMEM((16,), jnp.float32),
        pltpu.SemaphoreType.DMA,
    ],
)


def kernel(x, edge_index, adj_data, retina_scale, dm_idx, dm_vals, fc_w, fc_b):
    grid = 125
    eblk = E // grid
    rblk = NCHUNK // grid
    w2d, col2d, row2d = pl.pallas_call(
        _tc_fmt_body,
        grid=(grid,),
        in_specs=[pl.BlockSpec((2, eblk), lambda i: (0, i)),
                  pl.BlockSpec((eblk,), lambda i: (i,))],
        out_specs=[pl.BlockSpec((rblk, CH), lambda i: (i, 0)),
                   pl.BlockSpec((rblk, CH), lambda i: (i, 0)),
                   pl.BlockSpec((rblk, CH), lambda i: (i, 0))],
        out_shape=[jax.ShapeDtypeStruct((NCHUNK, CH), jnp.float32),
                   jax.ShapeDtypeStruct((NCHUNK, CH), jnp.int32),
                   jax.ShapeDtypeStruct((NCHUNK, CH), jnp.int32)],
    )(edge_index, adj_data)

    h0 = pl.pallas_call(
        _tc_h0_body,
        out_shape=jax.ShapeDtypeStruct((N,), jnp.float32),
    )(x.reshape(N), retina_scale)

    p0 = h0
    p1 = jnp.zeros((N,), jnp.float32)
    for _ in range(LAYERS):
        p0, p1, _hc = _sc_layer(p0, p1, col2d, row2d, w2d)

    dmi = jnp.zeros((KPAD,), jnp.int32).at[:K].set(dm_idx).reshape(KPAD // CH, CH)
    dmv = jnp.zeros((KPAD,), jnp.float32).at[:K].set(dm_vals).reshape(KPAD // CH, CH)
    fcw = jnp.zeros((KPAD,), jnp.float32).at[:K].set(fc_w[0]).reshape(KPAD // CH, CH)

    out_vec = _sc_final(p0, p1, dmi, dmv, fcw)
    return out_vec[0:1] + fc_b


# speedup vs baseline: 1.0327x; 1.0074x over previous
import jax
import jax.numpy as jnp
from jax import lax
from jax.experimental import pallas as pl
from jax.experimental.pallas import tpu as pltpu
from jax.experimental.pallas import tpu_sc as plsc

N = 100000
E = 3200000
K = 1000
LAYERS = 3

CH = 128
NCHUNK = E // CH
NTILES = 32
NSUPER = NCHUNK // 8
BASE_SUPER = NSUPER // NTILES
EXTRA = NSUPER - BASE_SUPER * NTILES
WINC = 24
NWIN = (BASE_SUPER * 8) // WINC
TAILC = BASE_SUPER * 8 - NWIN * WINC
SL = 6256
LAST_BASE = N - SL
SLA = 3072
SUBS = (3072, 3072, 112)
KPAD = 1024


def _tc_fmt_body(e_ref, a_ref, w_ref, col_ref, row_ref):
    blk = a_ref.shape[0] // CH
    w_ref[...] = jnp.log1p(a_ref[...]).reshape(blk, CH)
    row_ref[...] = e_ref[0, :].reshape(blk, CH)
    col_ref[...] = e_ref[1, :].reshape(blk, CH)


def _tc_h0_body(x_ref, r_ref, o_ref):
    o_ref[...] = x_ref[...] * r_ref[...]


def _sc_layer_body(h0_hbm, h1_hbm, col_hbm, row_hbm, w_hbm,
                   p0_hbm, p1_hbm, hc_hbm,
                   h_new_s, h_local, bufA, bufB,
                   col_b, row_b, w_b,
                   col_c, row_c, w_c,
                   ssem0, ssem1, lsem0, lsem1):
    c = lax.axis_index("c")
    s = lax.axis_index("s")
    wid = c * 16 + s

    base = jnp.minimum(s * SL, LAST_BASE)

    def _combine(off, size, n16):
        sla = pl.ds(0, size)
        pltpu.sync_copy(h0_hbm.at[pl.ds(off, size)], bufA.at[sla])
        pltpu.sync_copy(h1_hbm.at[pl.ds(off, size)], bufB.at[sla])

        def _add(j, carry):
            sl = pl.ds(j * 16, 16)
            bufA[sl] = bufA[sl] + bufB[sl]
            bufB[sl] = jnp.zeros((16,), jnp.float32)
            return carry

        lax.fori_loop(0, n16, _add, 0)
        pltpu.sync_copy(bufA.at[sla], hc_hbm.at[pl.ds(off, size)])
        pltpu.sync_copy(bufB.at[sla], h_new_s.at[pl.ds(off, size)])

    off = base
    for _size in SUBS:
        _combine(off, _size, _size // 16)
        off = off + _size
    plsc.subcore_barrier()
    pltpu.sync_copy(hc_hbm, h_local)

    cstart = wid * BASE_SUPER * 8
    bufs = ((col_b, row_b, w_b), (col_c, row_c, w_c))
    lsems = (lsem0, lsem1)
    ssems = (ssem0, ssem1)

    def _start_loads(p, k):
        cb, rb, wb = bufs[p]
        r0 = cstart + k * WINC
        ls = lsems[p]
        pltpu.async_copy(col_hbm.at[pl.ds(r0, WINC)], cb, ls)
        pltpu.async_copy(row_hbm.at[pl.ds(r0, WINC)], rb, ls)
        pltpu.async_copy(w_hbm.at[pl.ds(r0, WINC)], wb, ls)

    def _wait_loads(p):
        cb, rb, wb = bufs[p]
        ls = lsems[p]
        pltpu.make_async_copy(col_hbm.at[pl.ds(0, WINC)], cb, ls).wait()
        pltpu.make_async_copy(row_hbm.at[pl.ds(0, WINC)], rb, ls).wait()
        pltpu.make_async_copy(w_hbm.at[pl.ds(0, WINC)], wb, ls).wait()

    def _work(p, nch):
        cb, rb, wb = bufs[p]

        def _mf(i, cy):
            for j in range(8):
                sl = pl.ds(j * 16, 16)
                vals = plsc.load_gather(h_local, [cb[i, sl]])
                wb[i, sl] = vals * wb[i, sl]
            pltpu.async_copy(wb.at[i], h_new_s.at[rb.at[i]], ssems[p],
                             add=True)
            return cy

        lax.fori_loop(0, nch, _mf, 0)

    def _drain_scatters(p, nch):
        _, _, wb = bufs[p]
        pltpu.make_async_copy(w_hbm.at[pl.ds(0, nch)], wb.at[pl.ds(0, nch)],
                              ssems[p]).wait()

    _start_loads(0, 0)

    def _pair(t, carry):
        k0 = 2 * t
        _wait_loads(0)

        @pl.when(t > 0)
        def _dp1():
            _drain_scatters(1, WINC)

        _start_loads(1, k0 + 1)
        _work(0, WINC)
        _wait_loads(1)
        _drain_scatters(0, WINC)

        @pl.when(k0 + 2 < NWIN)
        def _sl():
            _start_loads(0, k0 + 2)

        _work(1, WINC)
        return carry

    lax.fori_loop(0, NWIN // 2, _pair, 0)
    _drain_scatters(1, WINC)

    def _do_window(r0, nch):
        sl_w = pl.ds(0, nch)
        pltpu.sync_copy(col_hbm.at[pl.ds(r0, nch)], col_b.at[sl_w])
        pltpu.sync_copy(row_hbm.at[pl.ds(r0, nch)], row_b.at[sl_w])
        pltpu.sync_copy(w_hbm.at[pl.ds(r0, nch)], w_b.at[sl_w])
        _work(0, nch)
        _drain_scatters(0, nch)

    _do_window(cstart + NWIN * WINC, TAILC)

    @pl.when(wid < EXTRA)
    def _extra():
        _do_window((NTILES * BASE_SUPER + wid) * 8, 8)

    plsc.subcore_barrier()

    def _emit(off, size):
        sla = pl.ds(0, size)
        pltpu.sync_copy(h_new_s.at[pl.ds(off, size)], bufA.at[sla])

        @pl.when(c == 0)
        def _w0():
            pltpu.sync_copy(bufA.at[sla], p0_hbm.at[pl.ds(off, size)])

        @pl.when(c == 1)
        def _w1():
            pltpu.sync_copy(bufA.at[sla], p1_hbm.at[pl.ds(off, size)])

    off = base
    for _size in SUBS:
        _emit(off, _size)
        off = off + _size


def _sc_final_body(p0_hbm, p1_hbm, dmi_hbm, dmv_hbm, fcw_hbm, out_hbm,
                   dmi_b, g0, g1, dv, fw, ob, gsem):
    c = lax.axis_index("c")
    s = lax.axis_index("s")

    @pl.when((c == 0) & (s == 0))
    def _work():
        pltpu.sync_copy(dmi_hbm, dmi_b)
        pltpu.sync_copy(dmv_hbm, dv)
        pltpu.sync_copy(fcw_hbm, fw)

        def _fire(i, cy):
            pltpu.async_copy(p0_hbm.at[dmi_b.at[i]], g0.at[i], gsem)
            pltpu.async_copy(p1_hbm.at[dmi_b.at[i]], g1.at[i], gsem)
            return cy

        lax.fori_loop(0, KPAD // CH, _fire, 0)
        pltpu.make_async_copy(dmv_hbm, g0, gsem).wait()
        pltpu.make_async_copy(dmv_hbm, g1, gsem).wait()

        def _red(f, acc):
            i = f // 8
            sl = pl.ds((f % 8) * 16, 16)
            return acc + (g0[i, sl] + g1[i, sl]) * dv[i, sl] * fw[i, sl]

        acc = lax.fori_loop(0, (KPAD // CH) * 8, _red,
                            jnp.zeros((16,), jnp.float32))
        dnums = lax.GatherDimensionNumbers(
            offset_dims=(), collapsed_slice_dims=(0,), start_index_map=(0,))
        for shift in (8, 4, 2, 1):
            perm = lax.iota(jnp.int32, 16) ^ shift
            acc = acc + lax.gather(
                acc, perm[:, None], dnums, (1,),
                mode=lax.GatherScatterMode.PROMISE_IN_BOUNDS)
        ob[...] = acc
        pltpu.sync_copy(ob, out_hbm)


_sc_mesh = plsc.VectorSubcoreMesh(core_axis_name="c", subcore_axis_name="s")

_sc_layer = pl.kernel(
    _sc_layer_body,
    out_type=(jax.ShapeDtypeStruct((N,), jnp.float32),
              jax.ShapeDtypeStruct((N,), jnp.float32),
              jax.ShapeDtypeStruct((N,), jnp.float32)),
    mesh=_sc_mesh,
    compiler_params=pltpu.CompilerParams(needs_layout_passes=False),
    scratch_types=[
        pltpu.VMEM_SHARED((N,), jnp.float32),
        pltpu.VMEM((N,), jnp.float32),
        pltpu.VMEM((SLA,), jnp.float32),
        pltpu.VMEM((SLA,), jnp.float32),
        pltpu.VMEM((WINC, CH), jnp.int32),
        pltpu.VMEM((WINC, CH), jnp.int32),
        pltpu.VMEM((WINC, CH), jnp.float32),
        pltpu.VMEM((WINC, CH), jnp.int32),
        pltpu.VMEM((WINC, CH), jnp.int32),
        pltpu.VMEM((WINC, CH), jnp.float32),
        pltpu.SemaphoreType.DMA,
        pltpu.SemaphoreType.DMA,
        pltpu.SemaphoreType.DMA,
        pltpu.SemaphoreType.DMA,
    ],
)

_sc_final = pl.kernel(
    _sc_final_body,
    out_type=jax.ShapeDtypeStruct((16,), jnp.float32),
    mesh=_sc_mesh,
    scratch_types=[
        pltpu.VMEM((KPAD // CH, CH), jnp.int32),
        pltpu.VMEM((KPAD // CH, CH), jnp.float32),
        pltpu.VMEM((KPAD // CH, CH), jnp.float32),
        pltpu.VMEM((KPAD // CH, CH), jnp.float32),
        pltpu.VMEM((KPAD // CH, CH), jnp.float32),
        pltpu.VMEM((16,), jnp.float32),
        pltpu.SemaphoreType.DMA,
    ],
)


def kernel(x, edge_index, adj_data, retina_scale, dm_idx, dm_vals, fc_w, fc_b):
    grid = 125
    eblk = E // grid
    rblk = NCHUNK // grid
    w2d, col2d, row2d = pl.pallas_call(
        _tc_fmt_body,
        grid=(grid,),
        in_specs=[pl.BlockSpec((2, eblk), lambda i: (0, i)),
                  pl.BlockSpec((eblk,), lambda i: (i,))],
        out_specs=[pl.BlockSpec((rblk, CH), lambda i: (i, 0)),
                   pl.BlockSpec((rblk, CH), lambda i: (i, 0)),
                   pl.BlockSpec((rblk, CH), lambda i: (i, 0))],
        out_shape=[jax.ShapeDtypeStruct((NCHUNK, CH), jnp.float32),
                   jax.ShapeDtypeStruct((NCHUNK, CH), jnp.int32),
                   jax.ShapeDtypeStruct((NCHUNK, CH), jnp.int32)],
    )(edge_index, adj_data)

    h0 = pl.pallas_call(
        _tc_h0_body,
        out_shape=jax.ShapeDtypeStruct((N,), jnp.float32),
    )(x.reshape(N), retina_scale)

    p0 = h0
    p1 = jnp.zeros((N,), jnp.float32)
    for _ in range(LAYERS):
        p0, p1, _hc = _sc_layer(p0, p1, col2d, row2d, w2d)

    dmi = jnp.zeros((KPAD,), jnp.int32).at[:K].set(dm_idx).reshape(KPAD // CH, CH)
    dmv = jnp.zeros((KPAD,), jnp.float32).at[:K].set(dm_vals).reshape(KPAD // CH, CH)
    fcw = jnp.zeros((KPAD,), jnp.float32).at[:K].set(fc_w[0]).reshape(KPAD // CH, CH)

    out_vec = _sc_final(p0, p1, dmi, dmv, fcw)
    return out_vec[0:1] + fc_b
